# baseline (speedup 1.0000x reference)
import jax
import jax.numpy as jnp
from jax.experimental import pallas as pl
from jax.experimental.pallas import tpu as pltpu

_H = 128
_N = 32
_L = 3
_G = 32


def _mm(a, b):
    return jnp.dot(a, b)


def _body(*refs):
    x_ref, t_ref, g_ref, cst_ref, wt_ref, wp_ref = refs[0:6]
    o_ref = refs[-1]
    w = refs[6:-1]

    G, N = x_ref.shape[0], x_ref.shape[1]
    GN = G * N

    x = x_ref[...]
    hb = cst_ref[...] + t_ref[...] * wt_ref[...] + g_ref[...] * wp_ref[...]
    h = jnp.broadcast_to(hb[:, None, :], (G, N, _H)).reshape(GN, _H)

    k = 0
    for _ in range(_L):
        (W1a, W1b, w1c, b1, W2, b2,
         Wn1h, Wn1a, bn1, Wn2, bn2,
         Wc1, bc1, wc2) = (r[...] for r in w[k:k + 14])
        k += 14

        d0 = x[:, :, 0, None] - x[:, None, :, 0]
        d1 = x[:, :, 1, None] - x[:, None, :, 1]
        d2c = x[:, :, 2, None] - x[:, None, :, 2]
        dist = jnp.sqrt((d0 * d0 + d1 * d1) + d2c * d2c)
        dist = dist.astype(jnp.bfloat16).astype(jnp.float32)

        A = _mm(h, W1a) + b1
        Bp = _mm(h, W1b)
        pre = (A.reshape(G, N, 1, _H)
               + Bp.reshape(G, 1, N, _H)
               + dist[..., None] * w1c.reshape(1, 1, 1, _H))
        e1 = jax.nn.silu(pre).reshape(GN * N, _H)
        msg = _mm(e1, W2) + b2

        msg_d = _mm(jax.nn.silu(A + Bp), W2) + b2

        agg = (jnp.sum(msg.reshape(G, N, N * _H), axis=1)
               .reshape(GN, _H) - msg_d)

        hn = _mm(
            jax.nn.silu(_mm(h, Wn1h) + _mm(agg, Wn1a) + bn1), Wn2
        ) + bn2
        h = h + hn

        c1 = jax.nn.silu(_mm(msg, Wc1) + bc1)
        cwm = _mm(c1, wc2).reshape(G, N, N)
        s = jnp.sum(cwm, axis=1)
        xc = jax.lax.dot_general(
            cwm, x, (((1,), (1,)), ((0,), (0,))),
            precision=jax.lax.Precision.HIGHEST)
        x = x + xc - x * s[:, :, None]

    Wo1, bo1, Wo2, bo2 = (r[...] for r in w[k:k + 4])
    noise = _mm(jax.nn.silu(_mm(h, Wo1) + bo1), Wo2) + bo2
    o_ref[...] = noise.reshape(G, N, 3)


def kernel(x, t, bandgap, params):
    B, N = x.shape[0], x.shape[1]
    H = _H

    ne, te, pe = params["node_emb"], params["time_emb"], params["prop_emb"]
    cst = (ne["W"][0] + ne["b"] + te["b"] + pe["b"]).reshape(1, H)
    wt = te["W"].reshape(1, H)
    wp = pe["W"].reshape(1, H)

    weights = []
    for lp in params["layers"]:
        W1 = lp["edge1"]["W"]
        weights += [
            W1[:H], W1[H:2 * H],
            W1[2 * H:2 * H + 1].astype(jnp.bfloat16).astype(jnp.float32),
            lp["edge1"]["b"].reshape(1, H),
            lp["edge2"]["W"], lp["edge2"]["b"].reshape(1, H),
            lp["node1"]["W"][:H], lp["node1"]["W"][H:],
            lp["node1"]["b"].reshape(1, H),
            lp["node2"]["W"], lp["node2"]["b"].reshape(1, H),
            lp["coord1"]["W"], lp["coord1"]["b"].reshape(1, H),
            lp["coord2"]["W"],
        ]
    weights += [
        params["out1"]["W"], params["out1"]["b"].reshape(1, H),
        params["out2"]["W"], params["out2"]["b"].reshape(1, 3),
    ]

    G = _G
    grid = (B // G,)

    def const_spec(a):
        nd = a.ndim
        return pl.BlockSpec(a.shape, lambda i: (0,) * nd)

    in_specs = [
        pl.BlockSpec((G, N, 3), lambda i: (i, 0, 0)),
        pl.BlockSpec((G, 1), lambda i: (i, 0)),
        pl.BlockSpec((G, 1), lambda i: (i, 0)),
        const_spec(cst), const_spec(wt), const_spec(wp),
    ] + [const_spec(a) for a in weights]

    out = pl.pallas_call(
        _body,
        grid=grid,
        in_specs=in_specs,
        out_specs=pl.BlockSpec((G, N, 3), lambda i: (i, 0, 0)),
        out_shape=jax.ShapeDtypeStruct((B, N, 3), jnp.float32),
        compiler_params=pltpu.CompilerParams(
            dimension_semantics=("parallel",)),
    )(x, t.reshape(B, 1), bandgap.reshape(B, 1), cst, wt, wp, *weights)
    return out


# speedup vs baseline: 1.2404x; 1.2404x over previous
import jax
import jax.numpy as jnp
from jax.experimental import pallas as pl
from jax.experimental.pallas import tpu as pltpu

_H = 128
_N = 32
_L = 3
_G = 16


def _silu(v):
    return v * (0.5 * jnp.tanh(0.5 * v) + 0.5)


def _mm(a, b):
    return jnp.dot(a, b)


def _body(*refs):
    x_ref, t_ref, g_ref, cst_ref, wt_ref, wp_ref = refs[0:6]
    o_ref = refs[-1]
    w = refs[6:-1]

    G, N = x_ref.shape[0], x_ref.shape[1]
    GN = G * N

    x = x_ref[...]
    hb = cst_ref[...] + t_ref[...] * wt_ref[...] + g_ref[...] * wp_ref[...]
    h = jnp.broadcast_to(hb[:, None, :], (G, N, _H)).reshape(GN, _H)

    k = 0
    for _ in range(_L):
        (W1a, W1b, w1c, b1, W2, b2,
         Wn1h, Wn1a, bn1, Wn2, bn2,
         Wc1, bc1, wc2) = (r[...] for r in w[k:k + 14])
        k += 14

        d0 = x[:, :, 0, None] - x[:, None, :, 0]
        d1 = x[:, :, 1, None] - x[:, None, :, 1]
        d2c = x[:, :, 2, None] - x[:, None, :, 2]
        dist = jnp.sqrt((d0 * d0 + d1 * d1) + d2c * d2c)
        dist = dist.astype(jnp.bfloat16).astype(jnp.float32)

        A = _mm(h, W1a) + b1
        Bp = _mm(h, W1b)
        pre = (A.reshape(G, N, 1, _H)
               + Bp.reshape(G, 1, N, _H)
               + dist[..., None] * w1c.reshape(1, 1, 1, _H))
        e1 = _silu(pre).reshape(GN * N, _H)
        msg = _mm(e1, W2) + b2

        msg_d = _mm(_silu(A + Bp), W2) + b2

        agg = (jnp.sum(msg.reshape(G, N, N * _H), axis=1)
               .reshape(GN, _H) - msg_d)

        hn = _mm(
            _silu(_mm(h, Wn1h) + _mm(agg, Wn1a) + bn1), Wn2
        ) + bn2
        h = h + hn

        c1 = _silu(_mm(msg, Wc1) + bc1)
        cwm = _mm(c1, wc2).reshape(G, N, N)
        s = jnp.sum(cwm, axis=1)
        xc = jax.lax.dot_general(
            cwm, x, (((1,), (1,)), ((0,), (0,))),
            precision=jax.lax.Precision.HIGHEST)
        x = x + xc - x * s[:, :, None]

    Wo1, bo1, Wo2, bo2 = (r[...] for r in w[k:k + 4])
    noise = _mm(_silu(_mm(h, Wo1) + bo1), Wo2) + bo2
    o_ref[...] = noise.reshape(G, N, 3)


def kernel(x, t, bandgap, params):
    B, N = x.shape[0], x.shape[1]
    H = _H

    ne, te, pe = params["node_emb"], params["time_emb"], params["prop_emb"]
    cst = (ne["W"][0] + ne["b"] + te["b"] + pe["b"]).reshape(1, H)
    wt = te["W"].reshape(1, H)
    wp = pe["W"].reshape(1, H)

    weights = []
    for lp in params["layers"]:
        W1 = lp["edge1"]["W"]
        weights += [
            W1[:H], W1[H:2 * H],
            W1[2 * H:2 * H + 1].astype(jnp.bfloat16).astype(jnp.float32),
            lp["edge1"]["b"].reshape(1, H),
            lp["edge2"]["W"], lp["edge2"]["b"].reshape(1, H),
            lp["node1"]["W"][:H], lp["node1"]["W"][H:],
            lp["node1"]["b"].reshape(1, H),
            lp["node2"]["W"], lp["node2"]["b"].reshape(1, H),
            lp["coord1"]["W"], lp["coord1"]["b"].reshape(1, H),
            lp["coord2"]["W"],
        ]
    weights += [
        params["out1"]["W"], params["out1"]["b"].reshape(1, H),
        params["out2"]["W"], params["out2"]["b"].reshape(1, 3),
    ]

    G = _G
    grid = (B // G,)

    def const_spec(a):
        nd = a.ndim
        return pl.BlockSpec(a.shape, lambda i: (0,) * nd)

    in_specs = [
        pl.BlockSpec((G, N, 3), lambda i: (i, 0, 0)),
        pl.BlockSpec((G, 1), lambda i: (i, 0)),
        pl.BlockSpec((G, 1), lambda i: (i, 0)),
        const_spec(cst), const_spec(wt), const_spec(wp),
    ] + [const_spec(a) for a in weights]

    out = pl.pallas_call(
        _body,
        grid=grid,
        in_specs=in_specs,
        out_specs=pl.BlockSpec((G, N, 3), lambda i: (i, 0, 0)),
        out_shape=jax.ShapeDtypeStruct((B, N, 3), jnp.float32),
        compiler_params=pltpu.CompilerParams(
            dimension_semantics=("parallel",)),
    )(x, t.reshape(B, 1), bandgap.reshape(B, 1), cst, wt, wp, *weights)
    return out


# speedup vs baseline: 1.2648x; 1.0197x over previous
import jax
import jax.numpy as jnp
from jax.experimental import pallas as pl
from jax.experimental.pallas import tpu as pltpu

_H = 128
_N = 32
_L = 3
_G = 16


def _silu(v):
    a = 0.5 * v
    return a * jnp.tanh(a) + a


def _mm(a, b):
    return jnp.dot(a, b)


def _body(*refs):
    x_ref, t_ref, g_ref, cst_ref, wt_ref, wp_ref = refs[0:6]
    o_ref = refs[-1]
    w = refs[6:-1]

    G, N = x_ref.shape[0], x_ref.shape[1]
    GN = G * N

    x = x_ref[...]
    hb = cst_ref[...] + t_ref[...] * wt_ref[...] + g_ref[...] * wp_ref[...]
    h = jnp.broadcast_to(hb[:, None, :], (G, N, _H)).reshape(GN, _H)

    k = 0
    for _ in range(_L):
        (W1a, W1b, w1c, b1, W2, b2,
         Wn1h, Wn1a, bn1, Wn2, bn2,
         Wc1, bc1, wc2) = (r[...] for r in w[k:k + 14])
        k += 14

        d0 = x[:, :, 0, None] - x[:, None, :, 0]
        d1 = x[:, :, 1, None] - x[:, None, :, 1]
        d2c = x[:, :, 2, None] - x[:, None, :, 2]
        dist = jnp.sqrt((d0 * d0 + d1 * d1) + d2c * d2c)
        dist = dist.astype(jnp.bfloat16).astype(jnp.float32)

        A = _mm(h, W1a) + b1
        Bp = _mm(h, W1b)
        pre = (A.reshape(G, N, 1, _H)
               + Bp.reshape(G, 1, N, _H)
               + dist[..., None] * w1c.reshape(1, 1, 1, _H))
        e1 = _silu(pre).reshape(GN * N, _H)
        msg = _mm(e1, W2) + b2

        msg_d = _mm(_silu(A + Bp), W2) + b2

        agg = (jnp.sum(msg.reshape(G, N, N * _H), axis=1)
               .reshape(GN, _H) - msg_d)

        hn = _mm(
            _silu(_mm(h, Wn1h) + _mm(agg, Wn1a) + bn1), Wn2
        ) + bn2
        h = h + hn

        c1 = _silu(_mm(msg, Wc1) + bc1)
        cwm = _mm(c1, wc2).reshape(G, N, N)
        s = jnp.sum(cwm, axis=1)
        xc = jax.lax.dot_general(
            cwm, x, (((1,), (1,)), ((0,), (0,))),
            precision=jax.lax.Precision.HIGHEST)
        x = x + xc - x * s[:, :, None]

    Wo1, bo1, Wo2, bo2 = (r[...] for r in w[k:k + 4])
    noise = _mm(_silu(_mm(h, Wo1) + bo1), Wo2) + bo2
    o_ref[...] = noise.reshape(G, N, 3)


def kernel(x, t, bandgap, params):
    B, N = x.shape[0], x.shape[1]
    H = _H

    ne, te, pe = params["node_emb"], params["time_emb"], params["prop_emb"]
    cst = (ne["W"][0] + ne["b"] + te["b"] + pe["b"]).reshape(1, H)
    wt = te["W"].reshape(1, H)
    wp = pe["W"].reshape(1, H)

    weights = []
    for lp in params["layers"]:
        W1 = lp["edge1"]["W"]
        weights += [
            W1[:H], W1[H:2 * H],
            W1[2 * H:2 * H + 1].astype(jnp.bfloat16).astype(jnp.float32),
            lp["edge1"]["b"].reshape(1, H),
            lp["edge2"]["W"], lp["edge2"]["b"].reshape(1, H),
            lp["node1"]["W"][:H], lp["node1"]["W"][H:],
            lp["node1"]["b"].reshape(1, H),
            lp["node2"]["W"], lp["node2"]["b"].reshape(1, H),
            lp["coord1"]["W"], lp["coord1"]["b"].reshape(1, H),
            lp["coord2"]["W"],
        ]
    weights += [
        params["out1"]["W"], params["out1"]["b"].reshape(1, H),
        params["out2"]["W"], params["out2"]["b"].reshape(1, 3),
    ]

    G = _G
    grid = (B // G,)

    def const_spec(a):
        nd = a.ndim
        return pl.BlockSpec(a.shape, lambda i: (0,) * nd)

    in_specs = [
        pl.BlockSpec((G, N, 3), lambda i: (i, 0, 0)),
        pl.BlockSpec((G, 1), lambda i: (i, 0)),
        pl.BlockSpec((G, 1), lambda i: (i, 0)),
        const_spec(cst), const_spec(wt), const_spec(wp),
    ] + [const_spec(a) for a in weights]

    out = pl.pallas_call(
        _body,
        grid=grid,
        in_specs=in_specs,
        out_specs=pl.BlockSpec((G, N, 3), lambda i: (i, 0, 0)),
        out_shape=jax.ShapeDtypeStruct((B, N, 3), jnp.float32),
        compiler_params=pltpu.CompilerParams(
            dimension_semantics=("parallel",)),
    )(x, t.reshape(B, 1), bandgap.reshape(B, 1), cst, wt, wp, *weights)
    return out


# speedup vs baseline: 1.2846x; 1.0157x over previous
import jax
import jax.numpy as jnp
from jax.experimental import pallas as pl
from jax.experimental.pallas import tpu as pltpu

_H = 128
_N = 32
_L = 3
_G = 16


def _silu(v):
    a = 0.5 * v
    return a * jnp.tanh(a) + a


def _mm(a, b):
    return jnp.dot(a, b)


def _body(*refs):
    x_ref, t_ref, g_ref, cst_ref, wt_ref, wp_ref = refs[0:6]
    o_ref = refs[-1]
    w = refs[6:-1]

    G, N = x_ref.shape[0], x_ref.shape[1]
    GN = G * N

    x = x_ref[...]
    hb = cst_ref[...] + t_ref[...] * wt_ref[...] + g_ref[...] * wp_ref[...]
    h = jnp.broadcast_to(hb[:, None, :], (G, N, _H)).reshape(GN, _H)

    k = 0
    for li in range(_L):
        (W1a, W1b, w1c, b1, W2, b2,
         Wn1h, Wn1a, bn1, Wn2, bn2,
         Wc1, bc1, wc2) = (r[...] for r in w[k:k + 14])
        k += 14

        d0 = x[:, :, 0, None] - x[:, None, :, 0]
        d1 = x[:, :, 1, None] - x[:, None, :, 1]
        d2c = x[:, :, 2, None] - x[:, None, :, 2]
        dist = jnp.sqrt((d0 * d0 + d1 * d1) + d2c * d2c)
        dist = dist.astype(jnp.bfloat16).astype(jnp.float32)

        if li == 0:
            Cg = _mm(hb, W1a) + b1 + _mm(hb, W1b)
            pre = (Cg.reshape(G, 1, 1, _H)
                   + dist[..., None] * w1c.reshape(1, 1, 1, _H))
            md = _mm(_silu(Cg), W2) + b2
            msg_d = (jnp.broadcast_to(md[:, None, :], (G, N, _H))
                     .reshape(GN, _H))
        else:
            A = _mm(h, W1a) + b1
            Bp = _mm(h, W1b)
            pre = (A.reshape(G, N, 1, _H)
                   + Bp.reshape(G, 1, N, _H)
                   + dist[..., None] * w1c.reshape(1, 1, 1, _H))
            msg_d = _mm(_silu(A + Bp), W2) + b2
        e1 = _silu(pre).reshape(GN * N, _H)
        msg = _mm(e1, W2) + b2

        agg = (jnp.sum(msg.reshape(G, N, N * _H), axis=1)
               .reshape(GN, _H) - msg_d)

        hn = _mm(
            _silu(_mm(h, Wn1h) + _mm(agg, Wn1a) + bn1), Wn2
        ) + bn2
        h = h + hn

        c1 = _silu(_mm(msg, Wc1) + bc1)
        cwm = _mm(c1, wc2).reshape(G, N, N)
        s = jnp.sum(cwm, axis=1)
        xc = jax.lax.dot_general(
            cwm, x, (((1,), (1,)), ((0,), (0,))),
            precision=jax.lax.Precision.HIGHEST)
        x = x + xc - x * s[:, :, None]

    Wo1, bo1, Wo2, bo2 = (r[...] for r in w[k:k + 4])
    noise = _mm(_silu(_mm(h, Wo1) + bo1), Wo2) + bo2
    o_ref[...] = noise.reshape(G, N, 3)


def kernel(x, t, bandgap, params):
    B, N = x.shape[0], x.shape[1]
    H = _H

    ne, te, pe = params["node_emb"], params["time_emb"], params["prop_emb"]
    cst = (ne["W"][0] + ne["b"] + te["b"] + pe["b"]).reshape(1, H)
    wt = te["W"].reshape(1, H)
    wp = pe["W"].reshape(1, H)

    weights = []
    for lp in params["layers"]:
        W1 = lp["edge1"]["W"]
        weights += [
            W1[:H], W1[H:2 * H],
            W1[2 * H:2 * H + 1].astype(jnp.bfloat16).astype(jnp.float32),
            lp["edge1"]["b"].reshape(1, H),
            lp["edge2"]["W"], lp["edge2"]["b"].reshape(1, H),
            lp["node1"]["W"][:H], lp["node1"]["W"][H:],
            lp["node1"]["b"].reshape(1, H),
            lp["node2"]["W"], lp["node2"]["b"].reshape(1, H),
            lp["coord1"]["W"], lp["coord1"]["b"].reshape(1, H),
            lp["coord2"]["W"],
        ]
    weights += [
        params["out1"]["W"], params["out1"]["b"].reshape(1, H),
        params["out2"]["W"], params["out2"]["b"].reshape(1, 3),
    ]

    G = _G
    grid = (B // G,)

    def const_spec(a):
        nd = a.ndim
        return pl.BlockSpec(a.shape, lambda i: (0,) * nd)

    in_specs = [
        pl.BlockSpec((G, N, 3), lambda i: (i, 0, 0)),
        pl.BlockSpec((G, 1), lambda i: (i, 0)),
        pl.BlockSpec((G, 1), lambda i: (i, 0)),
        const_spec(cst), const_spec(wt), const_spec(wp),
    ] + [const_spec(a) for a in weights]

    out = pl.pallas_call(
        _body,
        grid=grid,
        in_specs=in_specs,
        out_specs=pl.BlockSpec((G, N, 3), lambda i: (i, 0, 0)),
        out_shape=jax.ShapeDtypeStruct((B, N, 3), jnp.float32),
        compiler_params=pltpu.CompilerParams(
            dimension_semantics=("parallel",)),
    )(x, t.reshape(B, 1), bandgap.reshape(B, 1), cst, wt, wp, *weights)
    return out
